# Initial kernel scaffold; baseline (speedup 1.0000x reference)
import jax
import jax.numpy as jnp
from jax.experimental import pallas as pl


def kernel(x, edge_index, edge_weight, W1, b1, W2, b2):
    raise NotImplementedError("write your pallas kernel here")


# speedup vs baseline: 20.4346x; 20.4346x over previous
import functools

import jax
import jax.numpy as jnp
from jax import lax
from jax.experimental import pallas as pl
from jax.experimental.pallas import tpu as pltpu
from jax.experimental.pallas import tpu_sc as plsc

N = 10000
E = 320000
F_IN = 128
H = 16
C = 40

NC = 2
NS = 16
NW = NC * NS

NP = 10240
E_PAD = 327680
EROWS = E_PAD // 128
ROWS_PER_TILE = E_PAD // NW // 128
ZROWS = NP // NS

_MESH = plsc.VectorSubcoreMesh(
    core_axis_name="c", subcore_axis_name="s", num_cores=NC, num_subcores=NS)


def _zero_rows(rows_v, nchunk):
    z16 = jnp.zeros((16,), jnp.float32)

    def body(i, carry):
        for cch in range(nchunk):
            rows_v[i, pl.ds(cch * 16, 16)] = z16
        return carry

    lax.fori_loop(0, ZROWS, body, None)


def _deg_body(dst2d, ew2d, out, idx_v, ew_v, rows_v, acc_sh):
    cid = lax.axis_index("c")
    sid = lax.axis_index("s")
    wid = sid * NC + cid
    _zero_rows(rows_v, 1)
    pltpu.sync_copy(rows_v.at[pl.ds(0, ZROWS)],
                    acc_sh.at[pl.ds(sid * ZROWS, ZROWS)])
    plsc.subcore_barrier()

    for b in range(5):
        r0 = wid * ROWS_PER_TILE + b * 16
        pltpu.sync_copy(dst2d.at[pl.ds(r0, 16)], idx_v)
        pltpu.sync_copy(ew2d.at[pl.ds(r0, 16)], ew_v)

        def wcol(grp, carry):
            msk = lax.iota(jnp.int32, 16) == 0
            j = grp >> 3
            g = grp & 7
            ew16 = ew_v[j, pl.ds(g * 16, 16)]
            for t in range(16):
                sv = lax.broadcast_in_dim(ew16[t], (16,), ())
                rows_v[grp * 16 + t, :] = jnp.where(msk, sv, 0.0)
            return carry

        lax.fori_loop(0, 128, wcol, None)
        for j in range(16):
            pltpu.sync_copy(rows_v.at[pl.ds(j * 128, 128)],
                            acc_sh.at[idx_v.at[j]], add=True)
    plsc.subcore_barrier()
    pltpu.sync_copy(acc_sh.at[pl.ds(sid * ZROWS, ZROWS)],
                    out.at[cid].at[pl.ds(sid * ZROWS, ZROWS)])


def _edge_body(nchunk, nk, src2d, dst2d, ew2d, table, out,
               sidx, didx, ew_v, rows_v, acc_sh, gsem):
    cid = lax.axis_index("c")
    sid = lax.axis_index("s")
    wid = sid * NC + cid
    _zero_rows(rows_v, nchunk)
    pltpu.sync_copy(rows_v.at[pl.ds(0, ZROWS)],
                    acc_sh.at[pl.ds(sid * ZROWS, ZROWS)])
    plsc.subcore_barrier()

    nblk = ROWS_PER_TILE // nk
    nedge = nk * 128
    for b in range(nblk):
        r0 = wid * ROWS_PER_TILE + b * nk
        pltpu.sync_copy(src2d.at[pl.ds(r0, nk)], sidx)
        pltpu.sync_copy(dst2d.at[pl.ds(r0, nk)], didx)
        pltpu.sync_copy(ew2d.at[pl.ds(r0, nk)], ew_v)
        descs = [pltpu.async_copy(table.at[sidx.at[j]],
                                  rows_v.at[pl.ds(j * 128, 128)], gsem)
                 for j in range(nk)]
        for d in descs:
            d.wait()

        def mul(grp, carry):
            j = grp >> 3
            g = grp & 7
            ew16 = ew_v[j, pl.ds(g * 16, 16)]
            for t in range(16):
                e = grp * 16 + t
                sv = lax.broadcast_in_dim(ew16[t], (16,), ())
                for cch in range(nchunk):
                    rows_v[e, pl.ds(cch * 16, 16)] = (
                        rows_v[e, pl.ds(cch * 16, 16)] * sv)
            return carry

        lax.fori_loop(0, nedge // 16, mul, None)
        for j in range(nk):
            pltpu.sync_copy(rows_v.at[pl.ds(j * 128, 128)],
                            acc_sh.at[didx.at[j]], add=True)
    plsc.subcore_barrier()
    pltpu.sync_copy(acc_sh.at[pl.ds(sid * ZROWS, ZROWS)],
                    out.at[cid].at[pl.ds(sid * ZROWS, ZROWS)])


_SC_PARAMS = pltpu.CompilerParams(use_tc_tiling_on_sc=False)

_deg_call = pl.kernel(
    _deg_body,
    out_type=jax.ShapeDtypeStruct((NC, NP, 16), jnp.float32),
    mesh=_MESH,
    compiler_params=_SC_PARAMS,
    scratch_types=[
        pltpu.VMEM((16, 128), jnp.int32),
        pltpu.VMEM((16, 128), jnp.float32),
        pltpu.VMEM((2048, 16), jnp.float32),
        pltpu.VMEM_SHARED((NP, 16), jnp.float32),
    ],
)

_edge_call_16 = pl.kernel(
    functools.partial(_edge_body, 1, 16),
    out_type=jax.ShapeDtypeStruct((NC, NP, 16), jnp.float32),
    mesh=_MESH,
    compiler_params=_SC_PARAMS,
    scratch_types=[
        pltpu.VMEM((16, 128), jnp.int32),
        pltpu.VMEM((16, 128), jnp.int32),
        pltpu.VMEM((16, 128), jnp.float32),
        pltpu.VMEM((2048, 16), jnp.float32),
        pltpu.VMEM_SHARED((NP, 16), jnp.float32),
        pltpu.SemaphoreType.DMA,
    ],
)

_edge_call_48 = pl.kernel(
    functools.partial(_edge_body, 3, 8),
    out_type=jax.ShapeDtypeStruct((NC, NP, 48), jnp.float32),
    mesh=_MESH,
    compiler_params=_SC_PARAMS,
    scratch_types=[
        pltpu.VMEM((8, 128), jnp.int32),
        pltpu.VMEM((8, 128), jnp.int32),
        pltpu.VMEM((8, 128), jnp.float32),
        pltpu.VMEM((1024, 48), jnp.float32),
        pltpu.VMEM_SHARED((NP, 48), jnp.float32),
        pltpu.SemaphoreType.DMA,
    ],
)

_BLK = 1000
_GRID = N // _BLK


def _mm_body(x_ref, w_ref, o_ref):
    o_ref[...] = jnp.dot(x_ref[...], w_ref[...],
                         preferred_element_type=jnp.float32)


_mm_call = pl.pallas_call(
    _mm_body,
    grid=(_GRID,),
    in_specs=[
        pl.BlockSpec((_BLK, F_IN), lambda i: (i, 0)),
        pl.BlockSpec((F_IN, H), lambda i: (0, 0)),
    ],
    out_specs=pl.BlockSpec((_BLK, H), lambda i: (i, 0)),
    out_shape=jax.ShapeDtypeStruct((N, H), jnp.float32),
)


def _scale_body(degp_ref, xw_ref, y1_ref, dis_ref):
    deg = 1.0 + degp_ref[0, :, 0:1] + degp_ref[1, :, 0:1]
    dis = lax.rsqrt(deg)
    dis_ref[...] = dis
    y1_ref[...] = dis * xw_ref[...]


_scale_call = pl.pallas_call(
    _scale_body,
    grid=(_GRID,),
    in_specs=[
        pl.BlockSpec((NC, _BLK, 16), lambda i: (0, i, 0)),
        pl.BlockSpec((_BLK, H), lambda i: (i, 0)),
    ],
    out_specs=[
        pl.BlockSpec((_BLK, H), lambda i: (i, 0)),
        pl.BlockSpec((_BLK, 1), lambda i: (i, 0)),
    ],
    out_shape=[
        jax.ShapeDtypeStruct((N, H), jnp.float32),
        jax.ShapeDtypeStruct((N, 1), jnp.float32),
    ],
)


def _layer2_body(acc_ref, y1_ref, dis_ref, b1_ref, w2_ref, y2_ref):
    dis = dis_ref[...]
    agg = acc_ref[0] + acc_ref[1] + y1_ref[...]
    h = jnp.maximum(dis * agg + b1_ref[...], 0.0)
    hw2 = jnp.dot(h, w2_ref[...], preferred_element_type=jnp.float32)
    y2_ref[...] = dis * hw2


_layer2_call = pl.pallas_call(
    _layer2_body,
    grid=(_GRID,),
    in_specs=[
        pl.BlockSpec((NC, _BLK, 16), lambda i: (0, i, 0)),
        pl.BlockSpec((_BLK, H), lambda i: (i, 0)),
        pl.BlockSpec((_BLK, 1), lambda i: (i, 0)),
        pl.BlockSpec((1, H), lambda i: (0, 0)),
        pl.BlockSpec((H, 48), lambda i: (0, 0)),
    ],
    out_specs=pl.BlockSpec((_BLK, 48), lambda i: (i, 0)),
    out_shape=jax.ShapeDtypeStruct((N, 48), jnp.float32),
)


def _final_body(acc_ref, y2_ref, dis_ref, b2_ref, o_ref):
    dis = dis_ref[...]
    agg = acc_ref[0] + acc_ref[1] + y2_ref[...]
    z = dis * agg + b2_ref[...]
    m = jnp.max(z, axis=1, keepdims=True)
    ez = jnp.exp(z - m)
    ssum = jnp.sum(ez, axis=1, keepdims=True)
    o_ref[...] = (z - m - jnp.log(ssum))[:, :C]


_final_call = pl.pallas_call(
    _final_body,
    grid=(_GRID,),
    in_specs=[
        pl.BlockSpec((NC, _BLK, 48), lambda i: (0, i, 0)),
        pl.BlockSpec((_BLK, 48), lambda i: (i, 0)),
        pl.BlockSpec((_BLK, 1), lambda i: (i, 0)),
        pl.BlockSpec((1, 48), lambda i: (0, 0)),
    ],
    out_specs=pl.BlockSpec((_BLK, C), lambda i: (i, 0)),
    out_shape=jax.ShapeDtypeStruct((N, C), jnp.float32),
)


def kernel(x, edge_index, edge_weight, W1, b1, W2, b2):
    src = jnp.pad(edge_index[0], (0, E_PAD - E)).reshape(EROWS, 128)
    dst = jnp.pad(edge_index[1], (0, E_PAD - E)).reshape(EROWS, 128)
    ew = jnp.pad(edge_weight, (0, E_PAD - E)).reshape(EROWS, 128)

    xw1 = _mm_call(x, W1)
    deg_parts = _deg_call(dst, ew)
    y1, dis = _scale_call(deg_parts, xw1)

    acc1 = _edge_call_16(src, dst, ew, y1)

    b1r = b1.reshape(1, H)
    w2p = jnp.pad(W2, ((0, 0), (0, 48 - C)))
    y2 = _layer2_call(acc1, y1, dis, b1r, w2p)

    acc2 = _edge_call_48(src, dst, ew, y2)

    b2r = jnp.pad(b2, (0, 48 - C), constant_values=-1e30).reshape(1, 48)
    return _final_call(acc2, y2, dis, b2r)
